# baseline (speedup 1.0000x reference)
import functools

import jax
import jax.numpy as jnp
from jax import lax
from jax.experimental import pallas as pl
from jax.experimental.pallas import tpu as pltpu
from jax.experimental.pallas import tpu_sc as plsc

EMB_DIM = 64
NUM_CORES = 2
NUM_SUBCORES = 16
NUM_WORKERS = NUM_CORES * NUM_SUBCORES
CHUNK = 640


def _make_lookup(total_rows: int):
    chunks_per_worker = total_rows // (NUM_WORKERS * CHUNK)
    mesh = plsc.VectorSubcoreMesh(core_axis_name="c", subcore_axis_name="s")

    @functools.partial(
        pl.kernel,
        mesh=mesh,
        out_type=jax.ShapeDtypeStruct((total_rows, EMB_DIM), jnp.float32),
        scratch_types=[
            pltpu.VMEM((chunks_per_worker, CHUNK), jnp.int32),
            pltpu.VMEM((CHUNK, EMB_DIM), jnp.float32),
            pltpu.VMEM((CHUNK, EMB_DIM), jnp.float32),
            pltpu.SemaphoreType.DMA,
            pltpu.SemaphoreType.DMA,
            pltpu.SemaphoreType.DMA,
            pltpu.SemaphoreType.DMA,
        ],
        compiler_params=pltpu.CompilerParams(use_tc_tiling_on_sc=False),
    )
    def lookup(idx_hbm, table_hbm, out_hbm, idx_v, buf0, buf1, sg0, sg1, sw0, sw1):
        wid = lax.axis_index("s") * NUM_CORES + lax.axis_index("c")
        pltpu.sync_copy(idx_hbm.at[wid], idx_v)
        base = wid * chunks_per_worker * CHUNK

        bufs = [buf0, buf1]
        sg = [sg0, sg1]
        sw = [sw0, sw1]
        gathers = [None, None]
        writebacks = [None, None]
        for j in range(chunks_per_worker):
            b = j % 2
            if writebacks[b] is not None:
                writebacks[b].wait()
            gathers[b] = pltpu.async_copy(table_hbm.at[idx_v.at[j]], bufs[b], sg[b])
            if j >= 1:
                pb = (j - 1) % 2
                gathers[pb].wait()
                writebacks[pb] = pltpu.async_copy(
                    bufs[pb], out_hbm.at[pl.ds(base + (j - 1) * CHUNK, CHUNK)], sw[pb]
                )
        last = (chunks_per_worker - 1) % 2
        gathers[last].wait()
        writebacks[last] = pltpu.async_copy(
            bufs[last],
            out_hbm.at[pl.ds(base + (chunks_per_worker - 1) * CHUNK, CHUNK)],
            sw[last],
        )
        writebacks[1 - last].wait()
        writebacks[last].wait()

    return lookup


def kernel(x, table):
    batch, hist = x.shape
    total = batch * hist
    chunks_per_worker = total // (NUM_WORKERS * CHUNK)
    idx3d = x.reshape(NUM_WORKERS, chunks_per_worker, CHUNK)
    out = _make_lookup(total)(idx3d, table)
    return out.reshape(batch, hist, EMB_DIM)
